# Initial kernel scaffold; baseline (speedup 1.0000x reference)
import jax
import jax.numpy as jnp
from jax.experimental import pallas as pl


def kernel(x, table):
    raise NotImplementedError("write your pallas kernel here")


# speedup vs baseline: 1.7802x; 1.7802x over previous
import functools

import jax
import jax.numpy as jnp
from jax import lax
from jax.experimental import pallas as pl
from jax.experimental.pallas import tpu as pltpu
from jax.experimental.pallas import tpu_sc as plsc

IDX_W = 128
CHUNK = 512
NBUF = 2


@functools.lru_cache(maxsize=None)
def _make_gather(n, v, d):
    info = plsc.get_sparse_core_info()
    nw = info.num_cores * info.num_subcores
    per_w = n // nw
    assert n % nw == 0 and per_w % CHUNK == 0 and CHUNK % IDX_W == 0
    k = CHUNK // IDX_W
    g_total = per_w // CHUNK
    mesh = plsc.VectorSubcoreMesh(core_axis_name="c", subcore_axis_name="s")

    @functools.partial(
        pl.kernel,
        mesh=mesh,
        out_type=jax.ShapeDtypeStruct((n, d), jnp.float32),
        compiler_params=pltpu.CompilerParams(use_tc_tiling_on_sc=False),
        scratch_types=[
            pltpu.VMEM((NBUF, k, IDX_W), jnp.int32),
            pltpu.VMEM((NBUF, CHUNK, d), jnp.float32),
            pltpu.SemaphoreType.DMA,
            pltpu.SemaphoreType.DMA,
        ],
    )
    def gather(idx_hbm, table_hbm, out_hbm, idx_v, rows_v, sem0, sem1):
        wid = lax.axis_index("s") * info.num_cores + lax.axis_index("c")
        base = wid * per_w
        sems = (sem0, sem1)

        def fire(b, g):
            for j in range(k):
                pltpu.sync_copy(
                    idx_hbm.at[pl.ds(base + g * CHUNK + j * IDX_W, IDX_W)],
                    idx_v.at[b, j],
                )
            for j in range(k):
                pltpu.async_copy(
                    table_hbm.at[idx_v.at[b, j]],
                    rows_v.at[b, pl.ds(j * IDX_W, IDX_W)],
                    sems[b],
                )

        def drain_store(b, g):
            for j in range(k):
                pltpu.make_async_copy(
                    table_hbm.at[idx_v.at[b, j]],
                    rows_v.at[b, pl.ds(j * IDX_W, IDX_W)],
                    sems[b],
                ).wait()
            pltpu.sync_copy(
                rows_v.at[b],
                out_hbm.at[pl.ds(base + g * CHUNK, CHUNK)],
            )

        for b in range(NBUF):
            fire(b, b)

        def body(i, carry):
            g0 = i * NBUF
            for b in range(NBUF):
                drain_store(b, g0 + b)
                fire(b, g0 + b + NBUF)
            return carry

        lax.fori_loop(0, g_total // NBUF - 1, body, 0, unroll=False)
        for b in range(NBUF):
            drain_store(b, g_total - NBUF + b)

    return gather


def kernel(x, table):
    b, h = x.shape
    v, d = table.shape
    flat = x.reshape(b * h).astype(jnp.int32)
    out = _make_gather(b * h, v, d)(flat, table)
    return out.reshape(b, h, d)


# speedup vs baseline: 1.8755x; 1.0535x over previous
import functools

import jax
import jax.numpy as jnp
from jax import lax
from jax.experimental import pallas as pl
from jax.experimental.pallas import tpu as pltpu
from jax.experimental.pallas import tpu_sc as plsc

IDX_W = 128
CHUNK = 512
NBUF = 2


@functools.lru_cache(maxsize=None)
def _make_gather(n, v, d):
    info = plsc.get_sparse_core_info()
    nw = info.num_cores * info.num_subcores
    per_w = n // nw
    assert n % nw == 0 and per_w % CHUNK == 0 and CHUNK % IDX_W == 0
    k = CHUNK // IDX_W
    g_total = per_w // CHUNK
    rows_per_w = per_w // IDX_W
    mesh = plsc.VectorSubcoreMesh(core_axis_name="c", subcore_axis_name="s")

    @functools.partial(
        pl.kernel,
        mesh=mesh,
        out_type=jax.ShapeDtypeStruct((n, d), jnp.float32),
        compiler_params=pltpu.CompilerParams(use_tc_tiling_on_sc=False),
        scratch_types=[
            pltpu.VMEM((rows_per_w, IDX_W), jnp.int32),
            pltpu.VMEM((NBUF, CHUNK, d), jnp.float32),
            pltpu.SemaphoreType.DMA,
            pltpu.SemaphoreType.DMA,
        ],
    )
    def gather(idx_hbm, table_hbm, out_hbm, idx_v, rows_v, sem0, sem1):
        wid = lax.axis_index("s") * info.num_cores + lax.axis_index("c")
        base = wid * per_w
        sems = (sem0, sem1)

        pltpu.sync_copy(
            idx_hbm.at[pl.ds(wid * rows_per_w, rows_per_w)], idx_v
        )

        def fire(b, g):
            for j in range(k):
                pltpu.async_copy(
                    table_hbm.at[idx_v.at[g * k + j]],
                    rows_v.at[b, pl.ds(j * IDX_W, IDX_W)],
                    sems[b],
                )

        def drain_store(b, g):
            for j in range(k):
                pltpu.make_async_copy(
                    table_hbm.at[idx_v.at[g * k + j]],
                    rows_v.at[b, pl.ds(j * IDX_W, IDX_W)],
                    sems[b],
                ).wait()
            pltpu.sync_copy(
                rows_v.at[b],
                out_hbm.at[pl.ds(base + g * CHUNK, CHUNK)],
            )

        for b in range(NBUF):
            fire(b, b)

        def body(i, carry):
            g0 = i * NBUF
            for b in range(NBUF):
                drain_store(b, g0 + b)
                fire(b, g0 + b + NBUF)
            return carry

        lax.fori_loop(0, g_total // NBUF - 1, body, 0, unroll=False)
        for b in range(NBUF):
            drain_store(b, g_total - NBUF + b)

    return gather


def kernel(x, table):
    b, h = x.shape
    v, d = table.shape
    n = b * h
    flat = x.reshape(n // IDX_W, IDX_W).astype(jnp.int32)
    out = _make_gather(n, v, d)(flat, table)
    return out.reshape(b, h, d)
